# baseline (speedup 1.0000x reference)
import functools

import jax
import jax.numpy as jnp
from jax import lax
from jax.experimental import pallas as pl
from jax.experimental.pallas import tpu as pltpu
from jax.experimental.pallas import tpu_sc as plsc

LORA_SCALING = 2.0

NC = 2
NS = 16
NW = NC * NS

IDXV = 128
CHUNK = 1024
NSTREAM = CHUNK // IDXV


def _sc_gather_body(nchunk, x_hbm, w_hbm, a_hbm, base_hbm, arows_hbm,
                    idx_v, bufw, bufa, sem_i, sem_w, sem_a):
    wid = lax.axis_index("s") * NC + lax.axis_index("c")
    row0 = wid * (nchunk * NSTREAM)

    def chunk_body(c, carry):
        r = row0 + c * NSTREAM
        pltpu.async_copy(x_hbm.at[pl.ds(r, NSTREAM)], idx_v, sem_i).wait()
        descs = []
        for j in range(NSTREAM):
            descs.append(pltpu.async_copy(
                w_hbm.at[idx_v.at[j]], bufw.at[pl.ds(j * IDXV, IDXV)], sem_w))
            descs.append(pltpu.async_copy(
                a_hbm.at[idx_v.at[j]], bufa.at[pl.ds(j * IDXV, IDXV)], sem_a))
        for d in descs:
            d.wait()
        off = r * IDXV
        pltpu.sync_copy(bufw, base_hbm.at[pl.ds(off, CHUNK)])
        pltpu.sync_copy(bufa, arows_hbm.at[pl.ds(off, CHUNK)])
        return carry

    lax.fori_loop(0, nchunk, chunk_body, 0)


def _sc_gather(x2d, weight, lora_A):
    n = x2d.shape[0] * x2d.shape[1]
    nchunk = n // (NW * CHUNK)
    d = weight.shape[1]
    r = lora_A.shape[1]
    mesh = plsc.VectorSubcoreMesh(core_axis_name="c", subcore_axis_name="s",
                                  num_cores=NC, num_subcores=NS)
    kern = pl.kernel(
        functools.partial(_sc_gather_body, nchunk),
        out_type=(
            jax.ShapeDtypeStruct((n, d), jnp.float32),
            jax.ShapeDtypeStruct((n, r), jnp.float32),
        ),
        mesh=mesh,
        scratch_types=[
            pltpu.VMEM((NSTREAM, IDXV), jnp.int32),
            pltpu.VMEM((CHUNK, d), jnp.float32),
            pltpu.VMEM((CHUNK, r), jnp.float32),
            pltpu.SemaphoreType.DMA,
            pltpu.SemaphoreType.DMA,
            pltpu.SemaphoreType.DMA,
        ],
        compiler_params=pltpu.CompilerParams(use_tc_tiling_on_sc=False),
    )
    return kern(x2d, weight, lora_A)


def _tc_combine_body(base2_ref, arows8_ref, b_ref, out_ref):
    p2, _ = base2_ref.shape
    bblk = p2 * 2
    base = jnp.concatenate(
        [base2_ref[:, :64][:, None, :], base2_ref[:, 64:][:, None, :]],
        axis=1).reshape(bblk, 64)
    a8 = arows8_ref[...]
    arows = jnp.concatenate(
        [a8[:, 16 * k:16 * (k + 1)][:, None, :] for k in range(8)],
        axis=1).reshape(bblk, 16)
    delta = lax.dot_general(arows, b_ref[...],
                            (((1,), (0,)), ((), ())),
                            preferred_element_type=jnp.float32)
    res = base + delta * LORA_SCALING
    out_ref[0] = res.T


def _tc_combine_t(base, arows, lora_B, hist, batch):
    n, d = base.shape
    r = arows.shape[1]
    base2 = base.reshape(n // 2, 128)
    arows8 = arows.reshape(n * r // 128, 128)
    bblk = 2048
    nb = batch // bblk
    return pl.pallas_call(
        _tc_combine_body,
        grid=(hist, nb),
        in_specs=[
            pl.BlockSpec((bblk // 2, 128), lambda l, c: (l * nb + c, 0)),
            pl.BlockSpec((bblk * 16 // 128, 128), lambda l, c: (l * nb + c, 0)),
            pl.BlockSpec((r, d), lambda l, c: (0, 0)),
        ],
        out_specs=pl.BlockSpec((1, d, bblk), lambda l, c: (l, 0, c)),
        out_shape=jax.ShapeDtypeStruct((hist, d, batch), jnp.float32),
    )(base2, arows8, lora_B)


def kernel(x, weight, lora_A, lora_B):
    b, h = x.shape
    n = b * h
    d = weight.shape[1]
    xt2d = x.T.reshape(n // IDXV, IDXV).astype(jnp.int32)
    base, arows = _sc_gather(xt2d, weight, lora_A)
    out_t = _tc_combine_t(base, arows, lora_B, h, b)
    return jnp.transpose(out_t, (2, 0, 1))


# speedup vs baseline: 1.8282x; 1.8282x over previous
import functools

import jax
import jax.numpy as jnp
from jax import lax
from jax.experimental import pallas as pl
from jax.experimental.pallas import tpu as pltpu
from jax.experimental.pallas import tpu_sc as plsc

LORA_SCALING = 2.0

NC = 2
NS = 16
NW = NC * NS

IDXV = 128
CHUNK = 1024
NSTREAM = CHUNK // IDXV
BATCH = 16384
HALF = 1024


def _sc_gather_body(nchunk, x_hbm, xs_hbm, w_hbm, a_hbm, base2_hbm, arows_hbm,
                    idx_v, idxs_v, bufw, bufa, sem_i, sem_w, sem_a):
    wid = lax.axis_index("s") * NC + lax.axis_index("c")
    bpw = nchunk * CHUNK

    def chunk_body(c, carry):
        m0 = wid * bpw + c * CHUNK
        r = m0 // IDXV
        pltpu.async_copy(x_hbm.at[pl.ds(r, NSTREAM)], idx_v, sem_i).wait()
        pltpu.async_copy(xs_hbm.at[pl.ds(r, NSTREAM)], idxs_v, sem_i).wait()
        descs = []
        for j in range(NSTREAM):
            descs.append(pltpu.async_copy(
                w_hbm.at[idx_v.at[j]], bufw.at[pl.ds(j * IDXV, IDXV)], sem_w))
            descs.append(pltpu.async_copy(
                a_hbm.at[idxs_v.at[j]], bufa.at[pl.ds(j * IDXV, IDXV)], sem_a))
        for d in descs:
            d.wait()

        l = m0 // BATCH
        b0 = m0 - l * BATCH
        jh = (b0 // HALF) % 2
        row0 = l * (BATCH // 2) + (b0 // (2 * HALF)) * HALF
        pltpu.sync_copy(bufw, base2_hbm.at[pl.ds(row0, CHUNK), pl.ds(jh * 64, 64)])
        pltpu.sync_copy(bufa, arows_hbm.at[pl.ds(m0, CHUNK)])
        return carry

    lax.fori_loop(0, nchunk, chunk_body, 0)


def _sc_gather(x2d, xs2d, weight, lora_A):
    n = x2d.shape[0] * x2d.shape[1]
    nchunk = n // (NW * CHUNK)
    d = weight.shape[1]
    r = lora_A.shape[1]
    mesh = plsc.VectorSubcoreMesh(core_axis_name="c", subcore_axis_name="s",
                                  num_cores=NC, num_subcores=NS)
    kern = pl.kernel(
        functools.partial(_sc_gather_body, nchunk),
        out_type=(
            jax.ShapeDtypeStruct((n // 2, 2 * d), jnp.float32),
            jax.ShapeDtypeStruct((n, r), jnp.float32),
        ),
        mesh=mesh,
        scratch_types=[
            pltpu.VMEM((NSTREAM, IDXV), jnp.int32),
            pltpu.VMEM((NSTREAM, IDXV), jnp.int32),
            pltpu.VMEM((CHUNK, d), jnp.float32),
            pltpu.VMEM((CHUNK, r), jnp.float32),
            pltpu.SemaphoreType.DMA,
            pltpu.SemaphoreType.DMA,
            pltpu.SemaphoreType.DMA,
        ],
        compiler_params=pltpu.CompilerParams(use_tc_tiling_on_sc=False),
    )
    return kern(x2d, xs2d, weight, lora_A)


def _tc_combine_body(base2_ref, a8_ref, m8_ref, out_ref):
    bt = base2_ref[...].T
    delta8 = lax.dot_general(a8_ref[...], m8_ref[...],
                             (((1,), (0,)), ((), ())),
                             preferred_element_type=jnp.float32)
    d8t = delta8.T
    dt = jnp.concatenate([d8t[64 * u:64 * (u + 1), :] for u in range(8)],
                         axis=1)
    base_t = jnp.concatenate([bt[:64, :], bt[64:, :]], axis=1)
    out_ref[0] = base_t + dt


def _tc_combine_t(base2, arows, m8, hist, batch):
    d = 64
    r = 16
    n = arows.shape[0]
    a8 = arows.reshape(n * r // 128, 128)
    bblk = 2 * HALF
    nb = batch // bblk
    return pl.pallas_call(
        _tc_combine_body,
        grid=(hist, nb),
        in_specs=[
            pl.BlockSpec((HALF, 128), lambda l, c: (l * nb + c, 0)),
            pl.BlockSpec((bblk * r // 128, 128), lambda l, c: (l * nb + c, 0)),
            pl.BlockSpec((128, 8 * d), lambda l, c: (0, 0)),
        ],
        out_specs=pl.BlockSpec((1, d, bblk), lambda l, c: (l, 0, c)),
        out_shape=jax.ShapeDtypeStruct((hist, d, batch), jnp.float32),
    )(base2, a8, m8)


def kernel(x, weight, lora_A, lora_B):
    b, h = x.shape
    n = b * h
    d = weight.shape[1]
    xt = x.T.reshape(n).astype(jnp.int32)
    x2d = xt.reshape(n // IDXV, IDXV)
    xs2d = (xt.reshape(n // (2 * HALF), 8, 2 * HALF // 8)
            .transpose(0, 2, 1).reshape(n // IDXV, IDXV))
    base2, arows = _sc_gather(x2d, xs2d, weight, lora_A)
    m8 = jnp.kron(jnp.eye(8, dtype=weight.dtype), lora_B * LORA_SCALING)
    out_t = _tc_combine_t(base2, arows, m8, h, b)
    return jnp.transpose(out_t, (2, 0, 1))
